# baseline (speedup 1.0000x reference)
import functools

import jax
import jax.numpy as jnp
from jax import lax
from jax.experimental import pallas as pl
from jax.experimental.pallas import tpu as pltpu
from jax.experimental.pallas import tpu_sc as plsc

N = 10000
NP = 10112
RPT = 632
NB = 16
E = 320000
CH = 128
NCH = 2560
EP = NCH * CH
PAD_IDX = 10004
NC, NS = 2, 16
CPT_E = NCH // (NC * NS)
CH2 = 128
NCH2 = EP // CH2
CPT_E2 = NCH2 // (NC * NS)
CPT_F2 = NCH2 // NS
SPS = 40
NBUF = 2

_f32 = jnp.float32
_HI = lax.Precision.DEFAULT


def _sc_mesh():
    return plsc.VectorSubcoreMesh(core_axis_name="c", subcore_axis_name="s",
                                  num_cores=NC, num_subcores=NS)


def _fill_rows(buf, n_rows, row16):
    def body(i, carry):
        row_view = buf.at[i]
        for k in range(8):
            row_view[pl.ds(k * 16, 16)] = row16
        return carry

    lax.fori_loop(0, n_rows, body, 0)


def _copy_rows_to_spmem(buf, acc_sh, row0, bufrows):
    n_full = RPT // bufrows
    rem = RPT % bufrows
    for t in range(n_full):
        pltpu.sync_copy(buf, acc_sh.at[pl.ds(row0 + t * bufrows, bufrows)])
    if rem:
        pltpu.sync_copy(buf.at[pl.ds(0, rem)],
                        acc_sh.at[pl.ds(row0 + n_full * bufrows, rem)])


HCH = CH2 // 2


def _gather_chunk(ins_hbm, srcv, j, buf, gsem):
    for h in range(2):
        pltpu.async_copy(ins_hbm.at[srcv.at[j, pl.ds(h * HCH, HCH)]],
                         buf.at[pl.ds(h * HCH, HCH)], gsem)


def _wait_chunk(ins_hbm, srcv, j, buf, gsem):
    for h in range(2):
        pltpu.make_async_copy(ins_hbm.at[srcv.at[j, pl.ds(h * HCH, HCH)]],
                              buf.at[pl.ds(h * HCH, HCH)], gsem).wait()


def _edge_pipeline(ins_hbm, acc_sh, srcv, dstv, bufs, gsems, ssems,
                   n_chunks):
    for k in range(NBUF):
        _gather_chunk(ins_hbm, srcv, k, bufs[k], gsems[k])

    n_grp = n_chunks // NBUF

    def grp(q, carry):
        j0 = NBUF * q
        for k in range(NBUF):
            _wait_chunk(ins_hbm, srcv, j0 + k, bufs[k], gsems[k])
            pltpu.async_copy(bufs[k], acc_sh.at[dstv.at[j0 + k]], ssems[k],
                             add=True)
        for k in range(NBUF):
            pltpu.make_async_copy(bufs[k], acc_sh.at[dstv.at[j0 + k]],
                                  ssems[k]).wait()

            @pl.when(q < n_grp - 1)
            def _(k=k, j0=j0):
                _gather_chunk(ins_hbm, srcv, j0 + k + NBUF, bufs[k],
                              gsems[k])

        return carry

    lax.fori_loop(0, n_grp, grp, 0)


@functools.partial(
    pl.kernel,
    out_type=jax.ShapeDtypeStruct((NC * NP, 128), _f32),
    mesh=_sc_mesh(),
    scratch_types=[
        pltpu.VMEM((SPS, CH2), jnp.int32),
        pltpu.VMEM((SPS, CH2), jnp.int32),
        [pltpu.VMEM((CH2, 128), _f32)] * NBUF,
        pltpu.VMEM_SHARED((NP, 128), _f32),
        [pltpu.SemaphoreType.DMA] * NBUF,
        [pltpu.SemaphoreType.DMA] * NBUF,
    ],
    compiler_params=pltpu.CompilerParams(use_tc_tiling_on_sc=False),
)
def _spmm_edge(ins_hbm, src_hbm, dst_hbm, out_hbm, srcv, dstv, bufs,
               acc_sh, gsems, ssems):
    c = lax.axis_index("c")
    s = lax.axis_index("s")
    base = (c * NS + s) * CPT_E2

    @pl.when(c == 0)
    def _():
        pltpu.sync_copy(ins_hbm.at[pl.ds(s * RPT, RPT)],
                        acc_sh.at[pl.ds(s * RPT, RPT)])

    @pl.when(c == 1)
    def _():
        zero16 = jnp.zeros((16,), _f32)
        _fill_rows(bufs[0], CH2, zero16)
        _copy_rows_to_spmem(bufs[0], acc_sh, s * RPT, CH2)

    plsc.subcore_barrier()

    for st in range(CPT_E2 // SPS):
        pltpu.sync_copy(src_hbm.at[pl.ds(base + st * SPS, SPS)], srcv)
        pltpu.sync_copy(dst_hbm.at[pl.ds(base + st * SPS, SPS)], dstv)
        _edge_pipeline(ins_hbm, acc_sh, srcv, dstv, bufs, gsems, ssems, SPS)
    plsc.subcore_barrier()
    pltpu.sync_copy(acc_sh.at[pl.ds(s * RPT, RPT)],
                    out_hbm.at[pl.ds(c * NP + s * RPT, RPT)])


@functools.partial(
    pl.kernel,
    out_type=jax.ShapeDtypeStruct((NC * NP, 128), _f32),
    mesh=_sc_mesh(),
    scratch_types=[
        pltpu.VMEM((SPS, CH2), jnp.int32),
        pltpu.VMEM((SPS, CH2), jnp.int32),
        [pltpu.VMEM((CH2, 128), _f32)] * NBUF,
        pltpu.VMEM_SHARED((NP, 128), _f32),
        [pltpu.SemaphoreType.DMA] * NBUF,
        [pltpu.SemaphoreType.DMA] * NBUF,
    ],
    compiler_params=pltpu.CompilerParams(use_tc_tiling_on_sc=False),
)
def _spmm_feat(ins_hbm, src_hbm, dst_hbm, out_hbm, srcv, dstv, bufs,
               acc_sh, gsems, ssems):
    c = lax.axis_index("c")
    s = lax.axis_index("s")
    pltpu.sync_copy(ins_hbm.at[pl.ds(c * NP + s * RPT, RPT)],
                    acc_sh.at[pl.ds(s * RPT, RPT)])
    plsc.subcore_barrier()

    for st in range(CPT_F2 // SPS):
        base = s * CPT_F2 + st * SPS
        pltpu.sync_copy(src_hbm.at[c, pl.ds(base, SPS)], srcv)
        pltpu.sync_copy(dst_hbm.at[pl.ds(base, SPS)], dstv)
        _edge_pipeline(ins_hbm, acc_sh, srcv, dstv, bufs, gsems, ssems, SPS)
    plsc.subcore_barrier()
    pltpu.sync_copy(acc_sh.at[pl.ds(s * RPT, RPT)],
                    out_hbm.at[pl.ds(c * NP + s * RPT, RPT)])


@functools.partial(
    pl.kernel,
    out_type=jax.ShapeDtypeStruct((NC * NP, 128), _f32),
    mesh=_sc_mesh(),
    scratch_types=[
        pltpu.VMEM((CPT_E, CH), jnp.int32),
        pltpu.VMEM((CH, 128), _f32),
        pltpu.VMEM_SHARED((NP, 128), _f32),
        pltpu.SemaphoreType.DMA,
    ],
    compiler_params=pltpu.CompilerParams(use_tc_tiling_on_sc=False),
)
def _deg_kernel(dst_hbm, deg_hbm, dstv, rowb, acc_sh, sem):
    c = lax.axis_index("c")
    s = lax.axis_index("s")
    base = (c * NS + s) * CPT_E
    pltpu.sync_copy(dst_hbm.at[pl.ds(base, CPT_E)], dstv)

    one16 = jnp.ones((16,), _f32)
    zero16 = jnp.zeros((16,), _f32)
    init16 = jnp.where(c == 0, one16, zero16)
    _fill_rows(rowb, CH, init16)
    _copy_rows_to_spmem(rowb, acc_sh, s * RPT, CH)
    _fill_rows(rowb, CH, one16)
    plsc.subcore_barrier()

    GRP = 8

    def group(gi, carry):
        def fire(j, cc):
            pltpu.async_copy(rowb, acc_sh.at[dstv.at[gi * GRP + j]], sem,
                             add=True)
            return cc

        lax.fori_loop(0, GRP, fire, 0)

        def drain(j, cc):
            pltpu.make_async_copy(rowb, acc_sh.at[dstv.at[gi * GRP + j]],
                                  sem).wait()
            return cc

        lax.fori_loop(0, GRP, drain, 0)
        return carry

    lax.fori_loop(0, CPT_E // GRP, group, 0)
    plsc.subcore_barrier()
    pltpu.sync_copy(acc_sh.at[pl.ds(s * RPT, RPT)],
                    deg_hbm.at[pl.ds(c * NP + s * RPT, RPT)])


def _prep_body(x_ref, degp_ref, dinv_ref, ins_ref):
    deg = degp_ref[0][:, 0:1] + degp_ref[1][:, 0:1]
    dinv = lax.rsqrt(deg)
    dinv_ref[...] = jnp.broadcast_to(dinv, (RPT, 16))
    ins_ref[...] = x_ref[...] * dinv


def _tc_prep(x_pad, degp):
    return pl.pallas_call(
        _prep_body,
        grid=(NB,),
        in_specs=[
            pl.BlockSpec((RPT, 128), lambda i: (i, 0)),
            pl.BlockSpec((2, RPT, 128), lambda i: (0, i, 0)),
        ],
        out_specs=[
            pl.BlockSpec((RPT, 16), lambda i: (i, 0)),
            pl.BlockSpec((RPT, 128), lambda i: (i, 0)),
        ],
        out_shape=[
            jax.ShapeDtypeStruct((NP, 16), _f32),
            jax.ShapeDtypeStruct((NP, 128), _f32),
        ],
    )(x_pad, degp)


def _l1_body(acc_ref, dinv_ref, W1_ref, b1_ref, W2_ref, out_ref):
    a = acc_ref[0] + acc_ref[1]
    d = dinv_ref[...][:, 0:1]
    h = jnp.dot(a * d, W1_ref[...], preferred_element_type=_f32, precision=_HI)
    h = jnp.maximum(h + b1_ref[...], 0.0)
    y = jnp.dot(h, W2_ref[...], preferred_element_type=_f32, precision=_HI)
    y = y * d
    out_ref[0] = y[:, :128]
    out_ref[1] = y[:, 128:]


def _tc_l1(acc0, dinv, W1, b1, W2):
    return pl.pallas_call(
        _l1_body,
        grid=(NB,),
        in_specs=[
            pl.BlockSpec((2, RPT, 128), lambda i: (0, i, 0)),
            pl.BlockSpec((RPT, 16), lambda i: (i, 0)),
            pl.BlockSpec((128, 512), lambda i: (0, 0)),
            pl.BlockSpec((1, 512), lambda i: (0, 0)),
            pl.BlockSpec((512, 256), lambda i: (0, 0)),
        ],
        out_specs=pl.BlockSpec((2, RPT, 128), lambda i: (0, i, 0)),
        out_shape=jax.ShapeDtypeStruct((2, NP, 128), _f32),
    )(acc0, dinv, W1, b1, W2)


def _l2_body(acc_ref, dinv_ref, b2_ref, W3_ref, out_ref):
    a = jnp.concatenate([acc_ref[0], acc_ref[1]], axis=1)
    d = dinv_ref[...][:, 0:1]
    h = jnp.maximum(a * d + b2_ref[...], 0.0)
    y = jnp.dot(h, W3_ref[...], preferred_element_type=_f32, precision=_HI)
    out_ref[...] = y * d


def _tc_l2(acc1, dinv, b2, W3):
    return pl.pallas_call(
        _l2_body,
        grid=(NB,),
        in_specs=[
            pl.BlockSpec((2, RPT, 128), lambda i: (0, i, 0)),
            pl.BlockSpec((RPT, 16), lambda i: (i, 0)),
            pl.BlockSpec((1, 256), lambda i: (0, 0)),
            pl.BlockSpec((256, 128), lambda i: (0, 0)),
        ],
        out_specs=pl.BlockSpec((RPT, 128), lambda i: (i, 0)),
        out_shape=jax.ShapeDtypeStruct((NP, 128), _f32),
    )(acc1, dinv, b2, W3)


def _l3_body(acc_ref, dinv_ref, b3_ref, out_ref):
    a = acc_ref[0] + acc_ref[1]
    d = dinv_ref[...][:, 0:1]
    h = jnp.maximum(a * d + b3_ref[...], 0.0)
    out_ref[...] = h * d


def _tc_l3(acc2, dinv, b3):
    return pl.pallas_call(
        _l3_body,
        grid=(NB,),
        in_specs=[
            pl.BlockSpec((2, RPT, 128), lambda i: (0, i, 0)),
            pl.BlockSpec((RPT, 16), lambda i: (i, 0)),
            pl.BlockSpec((1, 128), lambda i: (0, 0)),
        ],
        out_specs=pl.BlockSpec((RPT, 128), lambda i: (i, 0)),
        out_shape=jax.ShapeDtypeStruct((NP, 128), _f32),
    )(acc2, dinv, b3)


def _head_body(acc_ref, dinv_ref, W4_ref, b4_ref, batch_ref, Wf1_ref,
               bf1_ref, Wf2_ref, bf2_ref, out_ref, sums, cnts):
    i = pl.program_id(0)
    a = acc_ref[0] + acc_ref[1]
    d = dinv_ref[...][:, 0:1]
    h = jnp.dot(a * d, W4_ref[...], preferred_element_type=_f32,
                precision=_HI)
    h = jnp.maximum(h + b4_ref[...], 0.0)
    bb = batch_ref[...][:, 0:1]
    oh = (bb == lax.broadcasted_iota(jnp.int32, (RPT, 16), 1)).astype(_f32)
    ssum = lax.dot_general(oh, h, (((0,), (0,)), ((), ())),
                           preferred_element_type=_f32, precision=_HI)
    scnt = lax.dot_general(oh, jnp.ones((RPT, 64), _f32),
                           (((0,), (0,)), ((), ())),
                           preferred_element_type=_f32, precision=_HI)

    @pl.when(i == 0)
    def _():
        sums[...] = ssum
        cnts[...] = scnt

    @pl.when(i > 0)
    def _():
        sums[...] += ssum
        cnts[...] += scnt

    @pl.when(i == NB - 1)
    def _():
        g = sums[...] / jnp.maximum(cnts[...], 1.0)
        g1 = jnp.dot(g, Wf1_ref[...], preferred_element_type=_f32,
                     precision=_HI)
        g1 = jnp.maximum(g1 + bf1_ref[...], 0.0)
        out_ref[...] = jnp.dot(g1, Wf2_ref[...], preferred_element_type=_f32,
                               precision=_HI) + bf2_ref[...]


def _tc_head(acc3, dinv, W4, b4, batch2d, Wf1, bf1, Wf2, bf2):
    return pl.pallas_call(
        _head_body,
        grid=(NB,),
        in_specs=[
            pl.BlockSpec((2, RPT, 128), lambda i: (0, i, 0)),
            pl.BlockSpec((RPT, 16), lambda i: (i, 0)),
            pl.BlockSpec((128, 64), lambda i: (0, 0)),
            pl.BlockSpec((1, 64), lambda i: (0, 0)),
            pl.BlockSpec((RPT, 16), lambda i: (i, 0)),
            pl.BlockSpec((64, 32), lambda i: (0, 0)),
            pl.BlockSpec((1, 32), lambda i: (0, 0)),
            pl.BlockSpec((32, 10), lambda i: (0, 0)),
            pl.BlockSpec((1, 10), lambda i: (0, 0)),
        ],
        out_specs=pl.BlockSpec((16, 10), lambda i: (0, 0)),
        out_shape=jax.ShapeDtypeStruct((16, 10), _f32),
        scratch_shapes=[
            pltpu.VMEM((16, 64), _f32),
            pltpu.VMEM((16, 64), _f32),
        ],
    )(acc3, dinv, W4, b4, batch2d, Wf1, bf1, Wf2, bf2)


def kernel(x, edge_index, batch, Wc1, bc1, Wc2, bc2, Wc3, bc3, Wc4, bc4,
           Wf1, bf1, Wf2, bf2):
    src = edge_index[0].astype(jnp.int32)
    dst = edge_index[1].astype(jnp.int32)
    pad = N + (jnp.arange(EP - E, dtype=jnp.int32) % (NP - N))
    src_flat = jnp.concatenate([src, pad])
    dst_flat = jnp.concatenate([dst, pad])
    dst_p = dst_flat.reshape(NCH, CH)
    src_p2 = src_flat.reshape(NCH2, CH2)
    dst_p2 = dst_flat.reshape(NCH2, CH2)
    src_both = jnp.stack([src_p2, src_p2 + NP])

    x_pad = jnp.pad(x, ((0, NP - N), (0, 0)))
    batch_p = jnp.concatenate(
        [batch.astype(jnp.int32), jnp.full((NP - N,), 16, jnp.int32)])
    batch2d = jnp.broadcast_to(batch_p[:, None], (NP, 16))

    degp = _deg_kernel(dst_p).reshape(2, NP, 128)
    dinv, ins0 = _tc_prep(x_pad, degp)

    acc0 = _spmm_edge(ins0, src_p2, dst_p2)
    ins1 = _tc_l1(acc0.reshape(2, NP, 128), dinv, Wc1,
                  bc1.reshape(1, 512), Wc2)
    acc1 = _spmm_feat(ins1.reshape(2 * NP, 128), src_both, dst_p2)
    ins2 = _tc_l2(acc1.reshape(2, NP, 128), dinv, bc2.reshape(1, 256), Wc3)
    acc2 = _spmm_edge(ins2, src_p2, dst_p2)
    ins3 = _tc_l3(acc2.reshape(2, NP, 128), dinv, bc3.reshape(1, 128))
    acc3 = _spmm_edge(ins3, src_p2, dst_p2)
    action = _tc_head(acc3.reshape(2, NP, 128), dinv, Wc4,
                      bc4.reshape(1, 64), batch2d, Wf1, bf1.reshape(1, 32),
                      Wf2, bf2.reshape(1, 10))
    return action


# speedup vs baseline: 1.1062x; 1.1062x over previous
import functools

import jax
import jax.numpy as jnp
from jax import lax
from jax.experimental import pallas as pl
from jax.experimental.pallas import tpu as pltpu
from jax.experimental.pallas import tpu_sc as plsc

N = 10000
NP = 10112
RPT = 632
NB = 16
E = 320000
CH = 128
NCH = 2560
EP = NCH * CH
PAD_IDX = 10004
NC, NS = 2, 16
CPT_E = NCH // (NC * NS)
CH2 = 128
NCH2 = EP // CH2
CPT_E2 = NCH2 // (NC * NS)
CPT_F2 = NCH2 // NS
SPS = 40
NBUF = 2

_f32 = jnp.float32
_HI = lax.Precision.DEFAULT


def _sc_mesh():
    return plsc.VectorSubcoreMesh(core_axis_name="c", subcore_axis_name="s",
                                  num_cores=NC, num_subcores=NS)


def _fill_rows(buf, n_rows, row16):
    def body(i, carry):
        row_view = buf.at[i]
        for k in range(8):
            row_view[pl.ds(k * 16, 16)] = row16
        return carry

    lax.fori_loop(0, n_rows, body, 0)


def _copy_rows_to_spmem(buf, acc_sh, row0, bufrows):
    n_full = RPT // bufrows
    rem = RPT % bufrows
    for t in range(n_full):
        pltpu.sync_copy(buf, acc_sh.at[pl.ds(row0 + t * bufrows, bufrows)])
    if rem:
        pltpu.sync_copy(buf.at[pl.ds(0, rem)],
                        acc_sh.at[pl.ds(row0 + n_full * bufrows, rem)])


HCH = CH2 // 2


def _gather_chunk(ins_hbm, srcv, j, buf, gsem):
    for h in range(2):
        pltpu.async_copy(ins_hbm.at[srcv.at[j, pl.ds(h * HCH, HCH)]],
                         buf.at[pl.ds(h * HCH, HCH)], gsem)


def _wait_chunk(ins_hbm, srcv, j, buf, gsem):
    for h in range(2):
        pltpu.make_async_copy(ins_hbm.at[srcv.at[j, pl.ds(h * HCH, HCH)]],
                              buf.at[pl.ds(h * HCH, HCH)], gsem).wait()


def _edge_pipeline(ins_hbm, acc_sh, srcv, dstv, bufs, gsems, ssems,
                   n_chunks):
    for k in range(NBUF):
        _gather_chunk(ins_hbm, srcv, k, bufs[k], gsems[k])

    n_grp = n_chunks // NBUF

    def grp(q, carry):
        j0 = NBUF * q
        for k in range(NBUF):
            _wait_chunk(ins_hbm, srcv, j0 + k, bufs[k], gsems[k])
            pltpu.async_copy(bufs[k], acc_sh.at[dstv.at[j0 + k]], ssems[k],
                             add=True)
        for k in range(NBUF):
            pltpu.make_async_copy(bufs[k], acc_sh.at[dstv.at[j0 + k]],
                                  ssems[k]).wait()

            @pl.when(q < n_grp - 1)
            def _(k=k, j0=j0):
                _gather_chunk(ins_hbm, srcv, j0 + k + NBUF, bufs[k],
                              gsems[k])

        return carry

    lax.fori_loop(0, n_grp, grp, 0)


def _make_spmm_edge(width, dtype, lanes):
    @functools.partial(
        pl.kernel,
        out_type=jax.ShapeDtypeStruct((NC * NP, width), dtype),
        mesh=_sc_mesh(),
        scratch_types=[
            pltpu.VMEM((SPS, CH2), jnp.int32),
            pltpu.VMEM((SPS, CH2), jnp.int32),
            [pltpu.VMEM((CH2, width), dtype)] * NBUF,
            pltpu.VMEM_SHARED((NP, width), dtype),
            [pltpu.SemaphoreType.DMA] * NBUF,
            [pltpu.SemaphoreType.DMA] * NBUF,
        ],
        compiler_params=pltpu.CompilerParams(use_tc_tiling_on_sc=False),
    )
    def spmm(ins_hbm, src_hbm, dst_hbm, out_hbm, srcv, dstv, bufs,
             acc_sh, gsems, ssems):
        c = lax.axis_index("c")
        s = lax.axis_index("s")
        base = (c * NS + s) * CPT_E2

        @pl.when(c == 0)
        def _():
            pltpu.sync_copy(ins_hbm.at[pl.ds(s * RPT, RPT)],
                            acc_sh.at[pl.ds(s * RPT, RPT)])

        @pl.when(c == 1)
        def _():
            zrow = jnp.zeros((lanes,), dtype)

            def body(i, carry):
                row_view = bufs[0].at[i]
                for k in range(width // lanes):
                    row_view[pl.ds(k * lanes, lanes)] = zrow
                return carry

            lax.fori_loop(0, CH2, body, 0)
            _copy_rows_to_spmem(bufs[0], acc_sh, s * RPT, CH2)

        plsc.subcore_barrier()

        for st in range(CPT_E2 // SPS):
            pltpu.sync_copy(src_hbm.at[pl.ds(base + st * SPS, SPS)], srcv)
            pltpu.sync_copy(dst_hbm.at[pl.ds(base + st * SPS, SPS)], dstv)
            _edge_pipeline(ins_hbm, acc_sh, srcv, dstv, bufs, gsems, ssems,
                           SPS)
        plsc.subcore_barrier()
        pltpu.sync_copy(acc_sh.at[pl.ds(s * RPT, RPT)],
                        out_hbm.at[pl.ds(c * NP + s * RPT, RPT)])

    return spmm


_spmm_edge = _make_spmm_edge(128, jnp.float32, 16)
_spmm_l2 = _make_spmm_edge(256, jnp.bfloat16, 32)


@functools.partial(
    pl.kernel,
    out_type=jax.ShapeDtypeStruct((NC * NP, 128), _f32),
    mesh=_sc_mesh(),
    scratch_types=[
        pltpu.VMEM((CPT_E, CH), jnp.int32),
        pltpu.VMEM((CH, 128), _f32),
        pltpu.VMEM_SHARED((NP, 128), _f32),
        pltpu.SemaphoreType.DMA,
    ],
    compiler_params=pltpu.CompilerParams(use_tc_tiling_on_sc=False),
)
def _deg_kernel(dst_hbm, deg_hbm, dstv, rowb, acc_sh, sem):
    c = lax.axis_index("c")
    s = lax.axis_index("s")
    base = (c * NS + s) * CPT_E
    pltpu.sync_copy(dst_hbm.at[pl.ds(base, CPT_E)], dstv)

    one16 = jnp.ones((16,), _f32)
    zero16 = jnp.zeros((16,), _f32)
    init16 = jnp.where(c == 0, one16, zero16)
    _fill_rows(rowb, CH, init16)
    _copy_rows_to_spmem(rowb, acc_sh, s * RPT, CH)
    _fill_rows(rowb, CH, one16)
    plsc.subcore_barrier()

    GRP = 8

    def group(gi, carry):
        def fire(j, cc):
            pltpu.async_copy(rowb, acc_sh.at[dstv.at[gi * GRP + j]], sem,
                             add=True)
            return cc

        lax.fori_loop(0, GRP, fire, 0)

        def drain(j, cc):
            pltpu.make_async_copy(rowb, acc_sh.at[dstv.at[gi * GRP + j]],
                                  sem).wait()
            return cc

        lax.fori_loop(0, GRP, drain, 0)
        return carry

    lax.fori_loop(0, CPT_E // GRP, group, 0)
    plsc.subcore_barrier()
    pltpu.sync_copy(acc_sh.at[pl.ds(s * RPT, RPT)],
                    deg_hbm.at[pl.ds(c * NP + s * RPT, RPT)])


def _prep_body(x_ref, degp_ref, dinv_ref, ins_ref):
    deg = degp_ref[0][:, 0:1] + degp_ref[1][:, 0:1]
    dinv = lax.rsqrt(deg)
    dinv_ref[...] = jnp.broadcast_to(dinv, (RPT, 16))
    ins_ref[...] = x_ref[...] * dinv


def _tc_prep(x_pad, degp):
    return pl.pallas_call(
        _prep_body,
        grid=(NB,),
        in_specs=[
            pl.BlockSpec((RPT, 128), lambda i: (i, 0)),
            pl.BlockSpec((2, RPT, 128), lambda i: (0, i, 0)),
        ],
        out_specs=[
            pl.BlockSpec((RPT, 16), lambda i: (i, 0)),
            pl.BlockSpec((RPT, 128), lambda i: (i, 0)),
        ],
        out_shape=[
            jax.ShapeDtypeStruct((NP, 16), _f32),
            jax.ShapeDtypeStruct((NP, 128), _f32),
        ],
    )(x_pad, degp)


def _l1_body(acc_ref, dinv_ref, W1_ref, b1_ref, W2_ref, out_ref):
    a = acc_ref[0] + acc_ref[1]
    d = dinv_ref[...][:, 0:1]
    h = jnp.dot(a * d, W1_ref[...], preferred_element_type=_f32, precision=_HI)
    h = jnp.maximum(h + b1_ref[...], 0.0)
    y = jnp.dot(h, W2_ref[...], preferred_element_type=_f32, precision=_HI)
    y = y * d
    out_ref[...] = y.astype(jnp.bfloat16)


def _tc_l1(acc0, dinv, W1, b1, W2):
    return pl.pallas_call(
        _l1_body,
        grid=(NB // 2,),
        in_specs=[
            pl.BlockSpec((2, 2 * RPT, 128), lambda i: (0, i, 0)),
            pl.BlockSpec((2 * RPT, 16), lambda i: (i, 0)),
            pl.BlockSpec((128, 512), lambda i: (0, 0)),
            pl.BlockSpec((1, 512), lambda i: (0, 0)),
            pl.BlockSpec((512, 256), lambda i: (0, 0)),
        ],
        out_specs=pl.BlockSpec((2 * RPT, 256), lambda i: (i, 0)),
        out_shape=jax.ShapeDtypeStruct((NP, 256), jnp.bfloat16),
    )(acc0, dinv, W1, b1, W2)


def _l2_body(acc_ref, dinv_ref, b2_ref, W3_ref, out_ref):
    a = (acc_ref[0].astype(_f32) + acc_ref[1].astype(_f32))
    d = dinv_ref[...][:, 0:1]
    h = jnp.maximum(a * d + b2_ref[...], 0.0)
    y = jnp.dot(h, W3_ref[...], preferred_element_type=_f32, precision=_HI)
    out_ref[...] = y * d


def _tc_l2(acc1, dinv, b2, W3):
    return pl.pallas_call(
        _l2_body,
        grid=(NB // 2,),
        in_specs=[
            pl.BlockSpec((2, 2 * RPT, 256), lambda i: (0, i, 0)),
            pl.BlockSpec((2 * RPT, 16), lambda i: (i, 0)),
            pl.BlockSpec((1, 256), lambda i: (0, 0)),
            pl.BlockSpec((256, 128), lambda i: (0, 0)),
        ],
        out_specs=pl.BlockSpec((2 * RPT, 128), lambda i: (i, 0)),
        out_shape=jax.ShapeDtypeStruct((NP, 128), _f32),
    )(acc1, dinv, b2, W3)


def _l3_body(acc_ref, dinv_ref, b3_ref, out_ref):
    a = acc_ref[0] + acc_ref[1]
    d = dinv_ref[...][:, 0:1]
    h = jnp.maximum(a * d + b3_ref[...], 0.0)
    out_ref[...] = h * d


def _tc_l3(acc2, dinv, b3):
    return pl.pallas_call(
        _l3_body,
        grid=(NB,),
        in_specs=[
            pl.BlockSpec((2, RPT, 128), lambda i: (0, i, 0)),
            pl.BlockSpec((RPT, 16), lambda i: (i, 0)),
            pl.BlockSpec((1, 128), lambda i: (0, 0)),
        ],
        out_specs=pl.BlockSpec((RPT, 128), lambda i: (i, 0)),
        out_shape=jax.ShapeDtypeStruct((NP, 128), _f32),
    )(acc2, dinv, b3)


def _head_body(acc_ref, dinv_ref, W4_ref, b4_ref, batch_ref, Wf1_ref,
               bf1_ref, Wf2_ref, bf2_ref, out_ref, sums, cnts):
    i = pl.program_id(0)
    a = acc_ref[0] + acc_ref[1]
    d = dinv_ref[...][:, 0:1]
    h = jnp.dot(a * d, W4_ref[...], preferred_element_type=_f32,
                precision=_HI)
    h = jnp.maximum(h + b4_ref[...], 0.0)
    bb = batch_ref[...][:, 0:1]
    oh = (bb == lax.broadcasted_iota(jnp.int32, (RPT, 16), 1)).astype(_f32)
    ssum = lax.dot_general(oh, h, (((0,), (0,)), ((), ())),
                           preferred_element_type=_f32, precision=_HI)
    scnt = lax.dot_general(oh, jnp.ones((RPT, 64), _f32),
                           (((0,), (0,)), ((), ())),
                           preferred_element_type=_f32, precision=_HI)

    @pl.when(i == 0)
    def _():
        sums[...] = ssum
        cnts[...] = scnt

    @pl.when(i > 0)
    def _():
        sums[...] += ssum
        cnts[...] += scnt

    @pl.when(i == NB - 1)
    def _():
        g = sums[...] / jnp.maximum(cnts[...], 1.0)
        g1 = jnp.dot(g, Wf1_ref[...], preferred_element_type=_f32,
                     precision=_HI)
        g1 = jnp.maximum(g1 + bf1_ref[...], 0.0)
        out_ref[...] = jnp.dot(g1, Wf2_ref[...], preferred_element_type=_f32,
                               precision=_HI) + bf2_ref[...]


def _tc_head(acc3, dinv, W4, b4, batch2d, Wf1, bf1, Wf2, bf2):
    return pl.pallas_call(
        _head_body,
        grid=(NB,),
        in_specs=[
            pl.BlockSpec((2, RPT, 128), lambda i: (0, i, 0)),
            pl.BlockSpec((RPT, 16), lambda i: (i, 0)),
            pl.BlockSpec((128, 64), lambda i: (0, 0)),
            pl.BlockSpec((1, 64), lambda i: (0, 0)),
            pl.BlockSpec((RPT, 16), lambda i: (i, 0)),
            pl.BlockSpec((64, 32), lambda i: (0, 0)),
            pl.BlockSpec((1, 32), lambda i: (0, 0)),
            pl.BlockSpec((32, 10), lambda i: (0, 0)),
            pl.BlockSpec((1, 10), lambda i: (0, 0)),
        ],
        out_specs=pl.BlockSpec((16, 10), lambda i: (0, 0)),
        out_shape=jax.ShapeDtypeStruct((16, 10), _f32),
        scratch_shapes=[
            pltpu.VMEM((16, 64), _f32),
            pltpu.VMEM((16, 64), _f32),
        ],
    )(acc3, dinv, W4, b4, batch2d, Wf1, bf1, Wf2, bf2)


def kernel(x, edge_index, batch, Wc1, bc1, Wc2, bc2, Wc3, bc3, Wc4, bc4,
           Wf1, bf1, Wf2, bf2):
    src = edge_index[0].astype(jnp.int32)
    dst = edge_index[1].astype(jnp.int32)
    pad = N + (jnp.arange(EP - E, dtype=jnp.int32) % (NP - N))
    src_flat = jnp.concatenate([src, pad])
    dst_flat = jnp.concatenate([dst, pad])
    dst_p = dst_flat.reshape(NCH, CH)
    src_p2 = src_flat.reshape(NCH2, CH2)
    dst_p2 = dst_flat.reshape(NCH2, CH2)

    x_pad = jnp.pad(x, ((0, NP - N), (0, 0)))
    batch_p = jnp.concatenate(
        [batch.astype(jnp.int32), jnp.full((NP - N,), 16, jnp.int32)])
    batch2d = jnp.broadcast_to(batch_p[:, None], (NP, 16))

    degp = _deg_kernel(dst_p).reshape(2, NP, 128)
    dinv, ins0 = _tc_prep(x_pad, degp)

    acc0 = _spmm_edge(ins0, src_p2, dst_p2)
    ins1 = _tc_l1(acc0.reshape(2, NP, 128), dinv, Wc1,
                  bc1.reshape(1, 512), Wc2)
    acc1 = _spmm_l2(ins1, src_p2, dst_p2)
    ins2 = _tc_l2(acc1.reshape(2, NP, 256), dinv, bc2.reshape(1, 256), Wc3)
    acc2 = _spmm_edge(ins2, src_p2, dst_p2)
    ins3 = _tc_l3(acc2.reshape(2, NP, 128), dinv, bc3.reshape(1, 128))
    acc3 = _spmm_edge(ins3, src_p2, dst_p2)
    action = _tc_head(acc3.reshape(2, NP, 128), dinv, Wc4,
                      bc4.reshape(1, 64), batch2d, Wf1, bf1.reshape(1, 32),
                      Wf2, bf2.reshape(1, 10))
    return action


# speedup vs baseline: 1.2516x; 1.1315x over previous
import functools

import jax
import jax.numpy as jnp
from jax import lax
from jax.experimental import pallas as pl
from jax.experimental.pallas import tpu as pltpu
from jax.experimental.pallas import tpu_sc as plsc

N = 10000
NP = 10112
RPT = 632
NB = 16
E = 320000
CH = 128
NCH = 2560
EP = NCH * CH
PAD_IDX = 10004
NC, NS = 2, 16
CPT_E = NCH // (NC * NS)
CH2 = 128
NCH2 = EP // CH2
CPT_E2 = NCH2 // (NC * NS)
CPT_F2 = NCH2 // NS
SPS = 40
NBUF = 2
RPB = 2 * RPT

_f32 = jnp.float32
_HI = lax.Precision.DEFAULT


def _sc_mesh():
    return plsc.VectorSubcoreMesh(core_axis_name="c", subcore_axis_name="s",
                                  num_cores=NC, num_subcores=NS)


def _fill_rows(buf, n_rows, row16):
    def body(i, carry):
        row_view = buf.at[i]
        for k in range(8):
            row_view[pl.ds(k * 16, 16)] = row16
        return carry

    lax.fori_loop(0, n_rows, body, 0)


def _copy_rows_to_spmem(buf, acc_sh, row0, bufrows):
    n_full = RPT // bufrows
    rem = RPT % bufrows
    for t in range(n_full):
        pltpu.sync_copy(buf, acc_sh.at[pl.ds(row0 + t * bufrows, bufrows)])
    if rem:
        pltpu.sync_copy(buf.at[pl.ds(0, rem)],
                        acc_sh.at[pl.ds(row0 + n_full * bufrows, rem)])


HCH = CH2 // 2


def _gather_chunk(ins_hbm, srcv, j, buf, gsem):
    for h in range(2):
        pltpu.async_copy(ins_hbm.at[srcv.at[j, pl.ds(h * HCH, HCH)]],
                         buf.at[pl.ds(h * HCH, HCH)], gsem)


def _wait_chunk(ins_hbm, srcv, j, buf, gsem):
    for h in range(2):
        pltpu.make_async_copy(ins_hbm.at[srcv.at[j, pl.ds(h * HCH, HCH)]],
                              buf.at[pl.ds(h * HCH, HCH)], gsem).wait()


def _edge_pipeline(ins_hbm, acc_sh, srcv, dstv, bufs, gsems, ssems,
                   n_chunks):
    for k in range(NBUF):
        _gather_chunk(ins_hbm, srcv, k, bufs[k], gsems[k])

    n_grp = n_chunks // NBUF

    def grp(q, carry):
        j0 = NBUF * q
        for k in range(NBUF):
            _wait_chunk(ins_hbm, srcv, j0 + k, bufs[k], gsems[k])
            pltpu.async_copy(bufs[k], acc_sh.at[dstv.at[j0 + k]], ssems[k],
                             add=True)
        for k in range(NBUF):
            pltpu.make_async_copy(bufs[k], acc_sh.at[dstv.at[j0 + k]],
                                  ssems[k]).wait()

            @pl.when(q < n_grp - 1)
            def _(k=k, j0=j0):
                _gather_chunk(ins_hbm, srcv, j0 + k + NBUF, bufs[k],
                              gsems[k])

        return carry

    lax.fori_loop(0, n_grp, grp, 0)


def _make_spmm_edge(width, dtype, lanes):
    @functools.partial(
        pl.kernel,
        out_type=jax.ShapeDtypeStruct((NC * NP, width), dtype),
        mesh=_sc_mesh(),
        scratch_types=[
            pltpu.VMEM((SPS, CH2), jnp.int32),
            pltpu.VMEM((SPS, CH2), jnp.int32),
            [pltpu.VMEM((CH2, width), dtype)] * NBUF,
            pltpu.VMEM_SHARED((NP, width), dtype),
            [pltpu.SemaphoreType.DMA] * NBUF,
            [pltpu.SemaphoreType.DMA] * NBUF,
        ],
        compiler_params=pltpu.CompilerParams(use_tc_tiling_on_sc=False),
    )
    def spmm(ins_hbm, src_hbm, dst_hbm, out_hbm, srcv, dstv, bufs,
             acc_sh, gsems, ssems):
        c = lax.axis_index("c")
        s = lax.axis_index("s")
        base = (c * NS + s) * CPT_E2

        @pl.when(c == 0)
        def _():
            pltpu.sync_copy(ins_hbm.at[pl.ds(s * RPT, RPT)],
                            acc_sh.at[pl.ds(s * RPT, RPT)])

        @pl.when(c == 1)
        def _():
            zrow = jnp.zeros((lanes,), dtype)

            def body(i, carry):
                row_view = bufs[0].at[i]
                for k in range(width // lanes):
                    row_view[pl.ds(k * lanes, lanes)] = zrow
                return carry

            lax.fori_loop(0, CH2, body, 0)
            _copy_rows_to_spmem(bufs[0], acc_sh, s * RPT, CH2)

        plsc.subcore_barrier()

        for st in range(CPT_E2 // SPS):
            pltpu.sync_copy(src_hbm.at[pl.ds(base + st * SPS, SPS)], srcv)
            pltpu.sync_copy(dst_hbm.at[pl.ds(base + st * SPS, SPS)], dstv)
            _edge_pipeline(ins_hbm, acc_sh, srcv, dstv, bufs, gsems, ssems,
                           SPS)
        plsc.subcore_barrier()
        pltpu.sync_copy(acc_sh.at[pl.ds(s * RPT, RPT)],
                        out_hbm.at[pl.ds(c * NP + s * RPT, RPT)])

    return spmm


_spmm_edge = _make_spmm_edge(128, jnp.bfloat16, 32)
_spmm_l2 = _make_spmm_edge(256, jnp.bfloat16, 32)


@functools.partial(
    pl.kernel,
    out_type=jax.ShapeDtypeStruct((NC * NP, 128), _f32),
    mesh=_sc_mesh(),
    scratch_types=[
        pltpu.VMEM((CPT_E, CH), jnp.int32),
        pltpu.VMEM((CH, 128), _f32),
        pltpu.VMEM_SHARED((NP, 128), _f32),
        pltpu.SemaphoreType.DMA,
    ],
    compiler_params=pltpu.CompilerParams(use_tc_tiling_on_sc=False),
)
def _deg_kernel(dst_hbm, deg_hbm, dstv, rowb, acc_sh, sem):
    c = lax.axis_index("c")
    s = lax.axis_index("s")
    base = (c * NS + s) * CPT_E
    pltpu.sync_copy(dst_hbm.at[pl.ds(base, CPT_E)], dstv)

    one16 = jnp.ones((16,), _f32)
    zero16 = jnp.zeros((16,), _f32)
    init16 = jnp.where(c == 0, one16, zero16)
    _fill_rows(rowb, CH, init16)
    _copy_rows_to_spmem(rowb, acc_sh, s * RPT, CH)
    _fill_rows(rowb, CH, one16)
    plsc.subcore_barrier()

    GRP = 8

    def group(gi, carry):
        def fire(j, cc):
            pltpu.async_copy(rowb, acc_sh.at[dstv.at[gi * GRP + j]], sem,
                             add=True)
            return cc

        lax.fori_loop(0, GRP, fire, 0)

        def drain(j, cc):
            pltpu.make_async_copy(rowb, acc_sh.at[dstv.at[gi * GRP + j]],
                                  sem).wait()
            return cc

        lax.fori_loop(0, GRP, drain, 0)
        return carry

    lax.fori_loop(0, CPT_E // GRP, group, 0)
    plsc.subcore_barrier()
    pltpu.sync_copy(acc_sh.at[pl.ds(s * RPT, RPT)],
                    deg_hbm.at[pl.ds(c * NP + s * RPT, RPT)])


def _prep_body(x_ref, degp_ref, dinv_ref, ins_ref):
    deg = degp_ref[0][:, 0:1] + degp_ref[1][:, 0:1]
    dinv = lax.rsqrt(deg)
    dinv_ref[...] = jnp.broadcast_to(dinv, (RPB, 16))
    ins_ref[...] = (x_ref[...] * dinv).astype(jnp.bfloat16)


def _tc_prep(x_pad, degp):
    return pl.pallas_call(
        _prep_body,
        grid=(NB // 2,),
        in_specs=[
            pl.BlockSpec((RPB, 128), lambda i: (i, 0)),
            pl.BlockSpec((2, RPB, 128), lambda i: (0, i, 0)),
        ],
        out_specs=[
            pl.BlockSpec((RPB, 16), lambda i: (i, 0)),
            pl.BlockSpec((RPB, 128), lambda i: (i, 0)),
        ],
        out_shape=[
            jax.ShapeDtypeStruct((NP, 16), _f32),
            jax.ShapeDtypeStruct((NP, 128), jnp.bfloat16),
        ],
    )(x_pad, degp)


def _l1_body(acc_ref, dinv_ref, W1_ref, b1_ref, W2_ref, out_ref):
    a = acc_ref[0].astype(_f32) + acc_ref[1].astype(_f32)
    d = dinv_ref[...][:, 0:1]
    h = jnp.dot(a * d, W1_ref[...], preferred_element_type=_f32, precision=_HI)
    h = jnp.maximum(h + b1_ref[...], 0.0)
    y = jnp.dot(h, W2_ref[...], preferred_element_type=_f32, precision=_HI)
    y = y * d
    out_ref[...] = y.astype(jnp.bfloat16)


def _tc_l1(acc0, dinv, W1, b1, W2):
    return pl.pallas_call(
        _l1_body,
        grid=(NB // 2,),
        in_specs=[
            pl.BlockSpec((2, RPB, 128), lambda i: (0, i, 0)),
            pl.BlockSpec((RPB, 16), lambda i: (i, 0)),
            pl.BlockSpec((128, 512), lambda i: (0, 0)),
            pl.BlockSpec((1, 512), lambda i: (0, 0)),
            pl.BlockSpec((512, 256), lambda i: (0, 0)),
        ],
        out_specs=pl.BlockSpec((RPB, 256), lambda i: (i, 0)),
        out_shape=jax.ShapeDtypeStruct((NP, 256), jnp.bfloat16),
    )(acc0, dinv, W1, b1, W2)


def _l2_body(acc_ref, dinv_ref, b2_ref, W3_ref, out_ref):
    a = (acc_ref[0].astype(_f32) + acc_ref[1].astype(_f32))
    d = dinv_ref[...][:, 0:1]
    h = jnp.maximum(a * d + b2_ref[...], 0.0)
    y = jnp.dot(h, W3_ref[...], preferred_element_type=_f32, precision=_HI)
    out_ref[...] = (y * d).astype(jnp.bfloat16)


def _tc_l2(acc1, dinv, b2, W3):
    return pl.pallas_call(
        _l2_body,
        grid=(NB // 2,),
        in_specs=[
            pl.BlockSpec((2, RPB, 256), lambda i: (0, i, 0)),
            pl.BlockSpec((RPB, 16), lambda i: (i, 0)),
            pl.BlockSpec((1, 256), lambda i: (0, 0)),
            pl.BlockSpec((256, 128), lambda i: (0, 0)),
        ],
        out_specs=pl.BlockSpec((RPB, 128), lambda i: (i, 0)),
        out_shape=jax.ShapeDtypeStruct((NP, 128), jnp.bfloat16),
    )(acc1, dinv, b2, W3)


def _l3_body(acc_ref, dinv_ref, b3_ref, out_ref):
    a = acc_ref[0].astype(_f32) + acc_ref[1].astype(_f32)
    d = dinv_ref[...][:, 0:1]
    h = jnp.maximum(a * d + b3_ref[...], 0.0)
    out_ref[...] = (h * d).astype(jnp.bfloat16)


def _tc_l3(acc2, dinv, b3):
    return pl.pallas_call(
        _l3_body,
        grid=(NB // 2,),
        in_specs=[
            pl.BlockSpec((2, RPB, 128), lambda i: (0, i, 0)),
            pl.BlockSpec((RPB, 16), lambda i: (i, 0)),
            pl.BlockSpec((1, 128), lambda i: (0, 0)),
        ],
        out_specs=pl.BlockSpec((RPB, 128), lambda i: (i, 0)),
        out_shape=jax.ShapeDtypeStruct((NP, 128), jnp.bfloat16),
    )(acc2, dinv, b3)


def _head_body(acc_ref, dinv_ref, W4_ref, b4_ref, batch_ref, Wf1_ref,
               bf1_ref, Wf2_ref, bf2_ref, out_ref, sums, cnts):
    i = pl.program_id(0)
    a = acc_ref[0].astype(_f32) + acc_ref[1].astype(_f32)
    d = dinv_ref[...][:, 0:1]
    h = jnp.dot(a * d, W4_ref[...], preferred_element_type=_f32,
                precision=_HI)
    h = jnp.maximum(h + b4_ref[...], 0.0)
    bb = batch_ref[...][:, 0:1]
    oh = (bb == lax.broadcasted_iota(jnp.int32, (RPB, 16), 1)).astype(_f32)
    ssum = lax.dot_general(oh, h, (((0,), (0,)), ((), ())),
                           preferred_element_type=_f32, precision=_HI)
    scnt = lax.dot_general(oh, jnp.ones((RPB, 64), _f32),
                           (((0,), (0,)), ((), ())),
                           preferred_element_type=_f32, precision=_HI)

    @pl.when(i == 0)
    def _():
        sums[...] = ssum
        cnts[...] = scnt

    @pl.when(i > 0)
    def _():
        sums[...] += ssum
        cnts[...] += scnt

    @pl.when(i == NB // 2 - 1)
    def _():
        g = sums[...] / jnp.maximum(cnts[...], 1.0)
        g1 = jnp.dot(g, Wf1_ref[...], preferred_element_type=_f32,
                     precision=_HI)
        g1 = jnp.maximum(g1 + bf1_ref[...], 0.0)
        out_ref[...] = jnp.dot(g1, Wf2_ref[...], preferred_element_type=_f32,
                               precision=_HI) + bf2_ref[...]


def _tc_head(acc3, dinv, W4, b4, batch2d, Wf1, bf1, Wf2, bf2):
    return pl.pallas_call(
        _head_body,
        grid=(NB // 2,),
        in_specs=[
            pl.BlockSpec((2, RPB, 128), lambda i: (0, i, 0)),
            pl.BlockSpec((RPB, 16), lambda i: (i, 0)),
            pl.BlockSpec((128, 64), lambda i: (0, 0)),
            pl.BlockSpec((1, 64), lambda i: (0, 0)),
            pl.BlockSpec((RPB, 16), lambda i: (i, 0)),
            pl.BlockSpec((64, 32), lambda i: (0, 0)),
            pl.BlockSpec((1, 32), lambda i: (0, 0)),
            pl.BlockSpec((32, 10), lambda i: (0, 0)),
            pl.BlockSpec((1, 10), lambda i: (0, 0)),
        ],
        out_specs=pl.BlockSpec((16, 10), lambda i: (0, 0)),
        out_shape=jax.ShapeDtypeStruct((16, 10), _f32),
        scratch_shapes=[
            pltpu.VMEM((16, 64), _f32),
            pltpu.VMEM((16, 64), _f32),
        ],
    )(acc3, dinv, W4, b4, batch2d, Wf1, bf1, Wf2, bf2)


def kernel(x, edge_index, batch, Wc1, bc1, Wc2, bc2, Wc3, bc3, Wc4, bc4,
           Wf1, bf1, Wf2, bf2):
    src = edge_index[0].astype(jnp.int32)
    dst = edge_index[1].astype(jnp.int32)
    pad = N + (jnp.arange(EP - E, dtype=jnp.int32) % (NP - N))
    src_flat = jnp.concatenate([src, pad])
    dst_flat = jnp.concatenate([dst, pad])
    dst_p = dst_flat.reshape(NCH, CH)
    src_p2 = src_flat.reshape(NCH2, CH2)
    dst_p2 = dst_flat.reshape(NCH2, CH2)

    x_pad = jnp.pad(x, ((0, NP - N), (0, 0)))
    batch_p = jnp.concatenate(
        [batch.astype(jnp.int32), jnp.full((NP - N,), 16, jnp.int32)])
    batch2d = jnp.broadcast_to(batch_p[:, None], (NP, 16))

    degp = _deg_kernel(dst_p).reshape(2, NP, 128)
    dinv, ins0 = _tc_prep(x_pad, degp)

    acc0 = _spmm_edge(ins0, src_p2, dst_p2)
    ins1 = _tc_l1(acc0.reshape(2, NP, 128), dinv, Wc1,
                  bc1.reshape(1, 512), Wc2)
    acc1 = _spmm_l2(ins1, src_p2, dst_p2)
    ins2 = _tc_l2(acc1.reshape(2, NP, 256), dinv, bc2.reshape(1, 256), Wc3)
    acc2 = _spmm_edge(ins2, src_p2, dst_p2)
    ins3 = _tc_l3(acc2.reshape(2, NP, 128), dinv, bc3.reshape(1, 128))
    acc3 = _spmm_edge(ins3, src_p2, dst_p2)
    action = _tc_head(acc3.reshape(2, NP, 128), dinv, Wc4,
                      bc4.reshape(1, 64), batch2d, Wf1, bf1.reshape(1, 32),
                      Wf2, bf2.reshape(1, 10))
    return action


# speedup vs baseline: 1.3857x; 1.1071x over previous
import functools

import jax
import jax.numpy as jnp
from jax import lax
from jax.experimental import pallas as pl
from jax.experimental.pallas import tpu as pltpu
from jax.experimental.pallas import tpu_sc as plsc

N = 10000
NP = 10112
RPT = 632
NB = 16
E = 320000
CH = 128
NCH = 2560
EP = NCH * CH
PAD_IDX = 10004
NC, NS = 2, 16
CPT_E = NCH // (NC * NS)
CH2 = 128
NCH2 = EP // CH2
CPT_E2 = NCH2 // (NC * NS)
CPT_F2 = NCH2 // NS
SPS = 40
RPB = 2 * RPT

_f32 = jnp.float32
_HI = lax.Precision.DEFAULT


def _sc_mesh():
    return plsc.VectorSubcoreMesh(core_axis_name="c", subcore_axis_name="s",
                                  num_cores=NC, num_subcores=NS)


def _fill_rows(buf, n_rows, row16):
    def body(i, carry):
        row_view = buf.at[i]
        for k in range(8):
            row_view[pl.ds(k * 16, 16)] = row16
        return carry

    lax.fori_loop(0, n_rows, body, 0)


def _copy_rows_to_spmem(buf, acc_sh, row0, bufrows):
    n_full = RPT // bufrows
    rem = RPT % bufrows
    for t in range(n_full):
        pltpu.sync_copy(buf, acc_sh.at[pl.ds(row0 + t * bufrows, bufrows)])
    if rem:
        pltpu.sync_copy(buf.at[pl.ds(0, rem)],
                        acc_sh.at[pl.ds(row0 + n_full * bufrows, rem)])


HCH = CH2 // 2


def _gather_chunk(ins_hbm, srcv, j, buf, gsem):
    for h in range(2):
        pltpu.async_copy(ins_hbm.at[srcv.at[j, pl.ds(h * HCH, HCH)]],
                         buf.at[pl.ds(h * HCH, HCH)], gsem)


def _wait_chunk(ins_hbm, srcv, j, buf, gsem):
    for h in range(2):
        pltpu.make_async_copy(ins_hbm.at[srcv.at[j, pl.ds(h * HCH, HCH)]],
                              buf.at[pl.ds(h * HCH, HCH)], gsem).wait()


def _edge_pipeline(ins_hbm, acc_sh, srcv, dstv, bufs, gsems, ssems,
                   n_chunks):
    nb = len(bufs)
    for k in range(nb):
        _gather_chunk(ins_hbm, srcv, k, bufs[k], gsems[k])

    n_grp = n_chunks // nb

    def grp(q, carry):
        j0 = nb * q
        for k in range(nb):
            _wait_chunk(ins_hbm, srcv, j0 + k, bufs[k], gsems[k])
            pltpu.async_copy(bufs[k], acc_sh.at[dstv.at[j0 + k]], ssems[k],
                             add=True)
        for k in range(nb):
            pltpu.make_async_copy(bufs[k], acc_sh.at[dstv.at[j0 + k]],
                                  ssems[k]).wait()

            @pl.when(q < n_grp - 1)
            def _(k=k, j0=j0):
                _gather_chunk(ins_hbm, srcv, j0 + k + nb, bufs[k],
                              gsems[k])

        return carry

    lax.fori_loop(0, n_grp, grp, 0)


def _make_spmm_edge(width, dtype, lanes, nbuf):
    @functools.partial(
        pl.kernel,
        out_type=jax.ShapeDtypeStruct((NC * NP, width), dtype),
        mesh=_sc_mesh(),
        scratch_types=[
            pltpu.VMEM((SPS, CH2), jnp.int32),
            pltpu.VMEM((SPS, CH2), jnp.int32),
            [pltpu.VMEM((CH2, width), dtype)] * nbuf,
            pltpu.VMEM_SHARED((NP, width), dtype),
            [pltpu.SemaphoreType.DMA] * nbuf,
            [pltpu.SemaphoreType.DMA] * nbuf,
        ],
        compiler_params=pltpu.CompilerParams(use_tc_tiling_on_sc=False),
    )
    def spmm(ins_hbm, src_hbm, dst_hbm, out_hbm, srcv, dstv, bufs,
             acc_sh, gsems, ssems):
        c = lax.axis_index("c")
        s = lax.axis_index("s")
        base = (c * NS + s) * CPT_E2

        @pl.when(c == 0)
        def _():
            pltpu.sync_copy(ins_hbm.at[pl.ds(s * RPT, RPT)],
                            acc_sh.at[pl.ds(s * RPT, RPT)])

        @pl.when(c == 1)
        def _():
            zrow = jnp.zeros((lanes,), dtype)

            def body(i, carry):
                row_view = bufs[0].at[i]
                for k in range(width // lanes):
                    row_view[pl.ds(k * lanes, lanes)] = zrow
                return carry

            lax.fori_loop(0, CH2, body, 0)
            _copy_rows_to_spmem(bufs[0], acc_sh, s * RPT, CH2)

        plsc.subcore_barrier()

        for st in range(CPT_E2 // SPS):
            pltpu.sync_copy(src_hbm.at[pl.ds(base + st * SPS, SPS)], srcv)
            pltpu.sync_copy(dst_hbm.at[pl.ds(base + st * SPS, SPS)], dstv)
            _edge_pipeline(ins_hbm, acc_sh, srcv, dstv, bufs, gsems, ssems,
                           SPS)
        plsc.subcore_barrier()
        pltpu.sync_copy(acc_sh.at[pl.ds(s * RPT, RPT)],
                        out_hbm.at[pl.ds(c * NP + s * RPT, RPT)])

    return spmm


_spmm_edge = _make_spmm_edge(128, jnp.bfloat16, 32, 4)
_spmm_l2 = _make_spmm_edge(256, jnp.bfloat16, 32, 2)


@functools.partial(
    pl.kernel,
    out_type=jax.ShapeDtypeStruct((NC * NP, 128), _f32),
    mesh=_sc_mesh(),
    scratch_types=[
        pltpu.VMEM((CPT_E, CH), jnp.int32),
        pltpu.VMEM((CH, 128), _f32),
        pltpu.VMEM_SHARED((NP, 128), _f32),
        pltpu.SemaphoreType.DMA,
    ],
    compiler_params=pltpu.CompilerParams(use_tc_tiling_on_sc=False),
)
def _deg_kernel(dst_hbm, deg_hbm, dstv, rowb, acc_sh, sem):
    c = lax.axis_index("c")
    s = lax.axis_index("s")
    base = (c * NS + s) * CPT_E
    pltpu.sync_copy(dst_hbm.at[pl.ds(base, CPT_E)], dstv)

    one16 = jnp.ones((16,), _f32)
    zero16 = jnp.zeros((16,), _f32)
    init16 = jnp.where(c == 0, one16, zero16)
    _fill_rows(rowb, CH, init16)
    _copy_rows_to_spmem(rowb, acc_sh, s * RPT, CH)
    _fill_rows(rowb, CH, one16)
    plsc.subcore_barrier()

    GRP = 8

    def group(gi, carry):
        def fire(j, cc):
            pltpu.async_copy(rowb, acc_sh.at[dstv.at[gi * GRP + j]], sem,
                             add=True)
            return cc

        lax.fori_loop(0, GRP, fire, 0)

        def drain(j, cc):
            pltpu.make_async_copy(rowb, acc_sh.at[dstv.at[gi * GRP + j]],
                                  sem).wait()
            return cc

        lax.fori_loop(0, GRP, drain, 0)
        return carry

    lax.fori_loop(0, CPT_E // GRP, group, 0)
    plsc.subcore_barrier()
    pltpu.sync_copy(acc_sh.at[pl.ds(s * RPT, RPT)],
                    deg_hbm.at[pl.ds(c * NP + s * RPT, RPT)])


def _prep_body(x_ref, degp_ref, dinv_ref, ins_ref):
    deg = degp_ref[0][:, 0:1] + degp_ref[1][:, 0:1]
    dinv = lax.rsqrt(deg)
    dinv_ref[...] = jnp.broadcast_to(dinv, (RPB, 16))
    ins_ref[...] = (x_ref[...] * dinv).astype(jnp.bfloat16)


def _tc_prep(x_pad, degp):
    return pl.pallas_call(
        _prep_body,
        grid=(NB // 2,),
        in_specs=[
            pl.BlockSpec((RPB, 128), lambda i: (i, 0)),
            pl.BlockSpec((2, RPB, 128), lambda i: (0, i, 0)),
        ],
        out_specs=[
            pl.BlockSpec((RPB, 16), lambda i: (i, 0)),
            pl.BlockSpec((RPB, 128), lambda i: (i, 0)),
        ],
        out_shape=[
            jax.ShapeDtypeStruct((NP, 16), _f32),
            jax.ShapeDtypeStruct((NP, 128), jnp.bfloat16),
        ],
    )(x_pad, degp)


def _l1_body(acc_ref, dinv_ref, W1_ref, b1_ref, W2_ref, out_ref):
    a = acc_ref[0].astype(_f32) + acc_ref[1].astype(_f32)
    d = dinv_ref[...][:, 0:1]
    h = jnp.dot(a * d, W1_ref[...], preferred_element_type=_f32, precision=_HI)
    h = jnp.maximum(h + b1_ref[...], 0.0)
    y = jnp.dot(h, W2_ref[...], preferred_element_type=_f32, precision=_HI)
    y = y * d
    out_ref[...] = y.astype(jnp.bfloat16)


def _tc_l1(acc0, dinv, W1, b1, W2):
    return pl.pallas_call(
        _l1_body,
        grid=(NB // 2,),
        in_specs=[
            pl.BlockSpec((2, RPB, 128), lambda i: (0, i, 0)),
            pl.BlockSpec((RPB, 16), lambda i: (i, 0)),
            pl.BlockSpec((128, 512), lambda i: (0, 0)),
            pl.BlockSpec((1, 512), lambda i: (0, 0)),
            pl.BlockSpec((512, 256), lambda i: (0, 0)),
        ],
        out_specs=pl.BlockSpec((RPB, 256), lambda i: (i, 0)),
        out_shape=jax.ShapeDtypeStruct((NP, 256), jnp.bfloat16),
    )(acc0, dinv, W1, b1, W2)


def _l2_body(acc_ref, dinv_ref, b2_ref, W3_ref, out_ref):
    a = (acc_ref[0].astype(_f32) + acc_ref[1].astype(_f32))
    d = dinv_ref[...][:, 0:1]
    h = jnp.maximum(a * d + b2_ref[...], 0.0)
    y = jnp.dot(h, W3_ref[...], preferred_element_type=_f32, precision=_HI)
    out_ref[...] = (y * d).astype(jnp.bfloat16)


def _tc_l2(acc1, dinv, b2, W3):
    return pl.pallas_call(
        _l2_body,
        grid=(NB // 2,),
        in_specs=[
            pl.BlockSpec((2, RPB, 256), lambda i: (0, i, 0)),
            pl.BlockSpec((RPB, 16), lambda i: (i, 0)),
            pl.BlockSpec((1, 256), lambda i: (0, 0)),
            pl.BlockSpec((256, 128), lambda i: (0, 0)),
        ],
        out_specs=pl.BlockSpec((RPB, 128), lambda i: (i, 0)),
        out_shape=jax.ShapeDtypeStruct((NP, 128), jnp.bfloat16),
    )(acc1, dinv, b2, W3)


def _l3_body(acc_ref, dinv_ref, b3_ref, out_ref):
    a = acc_ref[0].astype(_f32) + acc_ref[1].astype(_f32)
    d = dinv_ref[...][:, 0:1]
    h = jnp.maximum(a * d + b3_ref[...], 0.0)
    out_ref[...] = (h * d).astype(jnp.bfloat16)


def _tc_l3(acc2, dinv, b3):
    return pl.pallas_call(
        _l3_body,
        grid=(NB // 2,),
        in_specs=[
            pl.BlockSpec((2, RPB, 128), lambda i: (0, i, 0)),
            pl.BlockSpec((RPB, 16), lambda i: (i, 0)),
            pl.BlockSpec((1, 128), lambda i: (0, 0)),
        ],
        out_specs=pl.BlockSpec((RPB, 128), lambda i: (i, 0)),
        out_shape=jax.ShapeDtypeStruct((NP, 128), jnp.bfloat16),
    )(acc2, dinv, b3)


def _head_body(acc_ref, dinv_ref, W4_ref, b4_ref, batch_ref, Wf1_ref,
               bf1_ref, Wf2_ref, bf2_ref, out_ref, sums, cnts):
    i = pl.program_id(0)
    a = acc_ref[0].astype(_f32) + acc_ref[1].astype(_f32)
    d = dinv_ref[...][:, 0:1]
    h = jnp.dot(a * d, W4_ref[...], preferred_element_type=_f32,
                precision=_HI)
    h = jnp.maximum(h + b4_ref[...], 0.0)
    bb = batch_ref[...][:, 0:1]
    oh = (bb == lax.broadcasted_iota(jnp.int32, (RPB, 16), 1)).astype(_f32)
    ssum = lax.dot_general(oh, h, (((0,), (0,)), ((), ())),
                           preferred_element_type=_f32, precision=_HI)
    scnt = lax.dot_general(oh, jnp.ones((RPB, 64), _f32),
                           (((0,), (0,)), ((), ())),
                           preferred_element_type=_f32, precision=_HI)

    @pl.when(i == 0)
    def _():
        sums[...] = ssum
        cnts[...] = scnt

    @pl.when(i > 0)
    def _():
        sums[...] += ssum
        cnts[...] += scnt

    @pl.when(i == NB // 2 - 1)
    def _():
        g = sums[...] / jnp.maximum(cnts[...], 1.0)
        g1 = jnp.dot(g, Wf1_ref[...], preferred_element_type=_f32,
                     precision=_HI)
        g1 = jnp.maximum(g1 + bf1_ref[...], 0.0)
        out_ref[...] = jnp.dot(g1, Wf2_ref[...], preferred_element_type=_f32,
                               precision=_HI) + bf2_ref[...]


def _tc_head(acc3, dinv, W4, b4, batch2d, Wf1, bf1, Wf2, bf2):
    return pl.pallas_call(
        _head_body,
        grid=(NB // 2,),
        in_specs=[
            pl.BlockSpec((2, RPB, 128), lambda i: (0, i, 0)),
            pl.BlockSpec((RPB, 16), lambda i: (i, 0)),
            pl.BlockSpec((128, 64), lambda i: (0, 0)),
            pl.BlockSpec((1, 64), lambda i: (0, 0)),
            pl.BlockSpec((RPB, 16), lambda i: (i, 0)),
            pl.BlockSpec((64, 32), lambda i: (0, 0)),
            pl.BlockSpec((1, 32), lambda i: (0, 0)),
            pl.BlockSpec((32, 10), lambda i: (0, 0)),
            pl.BlockSpec((1, 10), lambda i: (0, 0)),
        ],
        out_specs=pl.BlockSpec((16, 10), lambda i: (0, 0)),
        out_shape=jax.ShapeDtypeStruct((16, 10), _f32),
        scratch_shapes=[
            pltpu.VMEM((16, 64), _f32),
            pltpu.VMEM((16, 64), _f32),
        ],
    )(acc3, dinv, W4, b4, batch2d, Wf1, bf1, Wf2, bf2)


def kernel(x, edge_index, batch, Wc1, bc1, Wc2, bc2, Wc3, bc3, Wc4, bc4,
           Wf1, bf1, Wf2, bf2):
    src = edge_index[0].astype(jnp.int32)
    dst = edge_index[1].astype(jnp.int32)
    pad = N + (jnp.arange(EP - E, dtype=jnp.int32) % (NP - N))
    src_flat = jnp.concatenate([src, pad])
    dst_flat = jnp.concatenate([dst, pad])
    dst_p = dst_flat.reshape(NCH, CH)
    src_p2 = src_flat.reshape(NCH2, CH2)
    dst_p2 = dst_flat.reshape(NCH2, CH2)

    x_pad = jnp.pad(x, ((0, NP - N), (0, 0)))
    batch_p = jnp.concatenate(
        [batch.astype(jnp.int32), jnp.full((NP - N,), 16, jnp.int32)])
    batch2d = jnp.broadcast_to(batch_p[:, None], (NP, 16))

    degp = _deg_kernel(dst_p).reshape(2, NP, 128)
    dinv, ins0 = _tc_prep(x_pad, degp)

    acc0 = _spmm_edge(ins0, src_p2, dst_p2)
    ins1 = _tc_l1(acc0.reshape(2, NP, 128), dinv, Wc1,
                  bc1.reshape(1, 512), Wc2)
    acc1 = _spmm_l2(ins1, src_p2, dst_p2)
    ins2 = _tc_l2(acc1.reshape(2, NP, 256), dinv, bc2.reshape(1, 256), Wc3)
    acc2 = _spmm_edge(ins2, src_p2, dst_p2)
    ins3 = _tc_l3(acc2.reshape(2, NP, 128), dinv, bc3.reshape(1, 128))
    acc3 = _spmm_edge(ins3, src_p2, dst_p2)
    action = _tc_head(acc3.reshape(2, NP, 128), dinv, Wc4,
                      bc4.reshape(1, 64), batch2d, Wf1, bf1.reshape(1, 32),
                      Wf2, bf2.reshape(1, 10))
    return action


# speedup vs baseline: 1.4244x; 1.0279x over previous
import functools

import jax
import jax.numpy as jnp
from jax import lax
from jax.experimental import pallas as pl
from jax.experimental.pallas import tpu as pltpu
from jax.experimental.pallas import tpu_sc as plsc

N = 10000
NP = 10112
RPT = 632
NB = 16
E = 320000
CH = 128
NCH = 2560
EP = NCH * CH
PAD_IDX = 10004
NC, NS = 2, 16
CPT_E = NCH // (NC * NS)
CH2 = 128
NCH2 = EP // CH2
CPT_E2 = NCH2 // (NC * NS)
CPT_F2 = NCH2 // NS
SPS = 40
RPB = 2 * RPT

_f32 = jnp.float32
_HI = lax.Precision.DEFAULT


def _sc_mesh():
    return plsc.VectorSubcoreMesh(core_axis_name="c", subcore_axis_name="s",
                                  num_cores=NC, num_subcores=NS)


def _fill_rows(buf, n_rows, row16):
    def body(i, carry):
        row_view = buf.at[i]
        for k in range(8):
            row_view[pl.ds(k * 16, 16)] = row16
        return carry

    lax.fori_loop(0, n_rows, body, 0)


def _copy_rows_to_spmem(buf, acc_sh, row0, bufrows):
    n_full = RPT // bufrows
    rem = RPT % bufrows
    for t in range(n_full):
        pltpu.sync_copy(buf, acc_sh.at[pl.ds(row0 + t * bufrows, bufrows)])
    if rem:
        pltpu.sync_copy(buf.at[pl.ds(0, rem)],
                        acc_sh.at[pl.ds(row0 + n_full * bufrows, rem)])


HCH = CH2 // 2


def _gather_chunk(ins_hbm, srcv, j, buf, gsem):
    for h in range(2):
        pltpu.async_copy(ins_hbm.at[srcv.at[j, pl.ds(h * HCH, HCH)]],
                         buf.at[pl.ds(h * HCH, HCH)], gsem)


def _wait_chunk(ins_hbm, srcv, j, buf, gsem):
    for h in range(2):
        pltpu.make_async_copy(ins_hbm.at[srcv.at[j, pl.ds(h * HCH, HCH)]],
                              buf.at[pl.ds(h * HCH, HCH)], gsem).wait()


def _edge_pipeline(ins_hbm, acc_sh, srcv, dstv, bufs, gsems, ssems,
                   n_chunks):
    nb = len(bufs)
    for k in range(nb):
        _gather_chunk(ins_hbm, srcv, k, bufs[k], gsems[k])

    n_grp = n_chunks // nb

    def grp(q, carry):
        j0 = nb * q
        for k in range(nb):
            _wait_chunk(ins_hbm, srcv, j0 + k, bufs[k], gsems[k])
            pltpu.async_copy(bufs[k], acc_sh.at[dstv.at[j0 + k]], ssems[k],
                             add=True)
        for k in range(nb):
            pltpu.make_async_copy(bufs[k], acc_sh.at[dstv.at[j0 + k]],
                                  ssems[k]).wait()

            @pl.when(q < n_grp - 1)
            def _(k=k, j0=j0):
                _gather_chunk(ins_hbm, srcv, j0 + k + nb, bufs[k],
                              gsems[k])

        return carry

    lax.fori_loop(0, n_grp, grp, 0)


def _make_spmm_edge(width, dtype, lanes, nbuf, sps):
    @functools.partial(
        pl.kernel,
        out_type=jax.ShapeDtypeStruct((NC * NP, width), dtype),
        mesh=_sc_mesh(),
        scratch_types=[
            pltpu.VMEM((sps, CH2), jnp.int32),
            pltpu.VMEM((sps, CH2), jnp.int32),
            [pltpu.VMEM((CH2, width), dtype)] * nbuf,
            pltpu.VMEM_SHARED((NP, width), dtype),
            [pltpu.SemaphoreType.DMA] * nbuf,
            [pltpu.SemaphoreType.DMA] * nbuf,
        ],
        compiler_params=pltpu.CompilerParams(use_tc_tiling_on_sc=False),
    )
    def spmm(ins_hbm, src_hbm, dst_hbm, out_hbm, srcv, dstv, bufs,
             acc_sh, gsems, ssems):
        c = lax.axis_index("c")
        s = lax.axis_index("s")
        base = (c * NS + s) * CPT_E2

        @pl.when(c == 0)
        def _():
            pltpu.sync_copy(ins_hbm.at[pl.ds(s * RPT, RPT)],
                            acc_sh.at[pl.ds(s * RPT, RPT)])

        @pl.when(c == 1)
        def _():
            zrow = jnp.zeros((lanes,), dtype)

            def body(i, carry):
                row_view = bufs[0].at[i]
                for k in range(width // lanes):
                    row_view[pl.ds(k * lanes, lanes)] = zrow
                return carry

            lax.fori_loop(0, CH2, body, 0)
            _copy_rows_to_spmem(bufs[0], acc_sh, s * RPT, CH2)

        plsc.subcore_barrier()

        for st in range(CPT_E2 // sps):
            pltpu.sync_copy(src_hbm.at[pl.ds(base + st * sps, sps)], srcv)
            pltpu.sync_copy(dst_hbm.at[pl.ds(base + st * sps, sps)], dstv)
            _edge_pipeline(ins_hbm, acc_sh, srcv, dstv, bufs, gsems, ssems,
                           sps)
        plsc.subcore_barrier()
        pltpu.sync_copy(acc_sh.at[pl.ds(s * RPT, RPT)],
                        out_hbm.at[pl.ds(c * NP + s * RPT, RPT)])

    return spmm


_spmm_edge = _make_spmm_edge(128, jnp.bfloat16, 32, 8, 80)
_spmm_l2 = _make_spmm_edge(256, jnp.bfloat16, 32, 2, SPS)


@functools.partial(
    pl.kernel,
    out_type=jax.ShapeDtypeStruct((NC * NP, 128), _f32),
    mesh=_sc_mesh(),
    scratch_types=[
        pltpu.VMEM((CPT_E, CH), jnp.int32),
        pltpu.VMEM((CH, 128), _f32),
        pltpu.VMEM_SHARED((NP, 128), _f32),
        pltpu.SemaphoreType.DMA,
    ],
    compiler_params=pltpu.CompilerParams(use_tc_tiling_on_sc=False),
)
def _deg_kernel(dst_hbm, deg_hbm, dstv, rowb, acc_sh, sem):
    c = lax.axis_index("c")
    s = lax.axis_index("s")
    base = (c * NS + s) * CPT_E
    pltpu.sync_copy(dst_hbm.at[pl.ds(base, CPT_E)], dstv)

    one16 = jnp.ones((16,), _f32)
    zero16 = jnp.zeros((16,), _f32)
    init16 = jnp.where(c == 0, one16, zero16)
    _fill_rows(rowb, CH, init16)
    _copy_rows_to_spmem(rowb, acc_sh, s * RPT, CH)
    _fill_rows(rowb, CH, one16)
    plsc.subcore_barrier()

    GRP = 8

    def group(gi, carry):
        def fire(j, cc):
            pltpu.async_copy(rowb, acc_sh.at[dstv.at[gi * GRP + j]], sem,
                             add=True)
            return cc

        lax.fori_loop(0, GRP, fire, 0)

        def drain(j, cc):
            pltpu.make_async_copy(rowb, acc_sh.at[dstv.at[gi * GRP + j]],
                                  sem).wait()
            return cc

        lax.fori_loop(0, GRP, drain, 0)
        return carry

    lax.fori_loop(0, CPT_E // GRP, group, 0)
    plsc.subcore_barrier()
    pltpu.sync_copy(acc_sh.at[pl.ds(s * RPT, RPT)],
                    deg_hbm.at[pl.ds(c * NP + s * RPT, RPT)])


def _prep_body(x_ref, degp_ref, dinv_ref, ins_ref):
    deg = degp_ref[0][:, 0:1] + degp_ref[1][:, 0:1]
    dinv = lax.rsqrt(deg)
    dinv_ref[...] = jnp.broadcast_to(dinv, (RPB, 16))
    ins_ref[...] = (x_ref[...] * dinv).astype(jnp.bfloat16)


def _tc_prep(x_pad, degp):
    return pl.pallas_call(
        _prep_body,
        grid=(NB // 2,),
        in_specs=[
            pl.BlockSpec((RPB, 128), lambda i: (i, 0)),
            pl.BlockSpec((2, RPB, 128), lambda i: (0, i, 0)),
        ],
        out_specs=[
            pl.BlockSpec((RPB, 16), lambda i: (i, 0)),
            pl.BlockSpec((RPB, 128), lambda i: (i, 0)),
        ],
        out_shape=[
            jax.ShapeDtypeStruct((NP, 16), _f32),
            jax.ShapeDtypeStruct((NP, 128), jnp.bfloat16),
        ],
    )(x_pad, degp)


def _l1_body(acc_ref, dinv_ref, W1_ref, b1_ref, W2_ref, out_ref):
    a = acc_ref[0].astype(_f32) + acc_ref[1].astype(_f32)
    d = dinv_ref[...][:, 0:1]
    h = jnp.dot(a * d, W1_ref[...], preferred_element_type=_f32, precision=_HI)
    h = jnp.maximum(h + b1_ref[...], 0.0)
    y = jnp.dot(h, W2_ref[...], preferred_element_type=_f32, precision=_HI)
    y = y * d
    out_ref[...] = y.astype(jnp.bfloat16)


def _tc_l1(acc0, dinv, W1, b1, W2):
    return pl.pallas_call(
        _l1_body,
        grid=(NB // 2,),
        in_specs=[
            pl.BlockSpec((2, RPB, 128), lambda i: (0, i, 0)),
            pl.BlockSpec((RPB, 16), lambda i: (i, 0)),
            pl.BlockSpec((128, 512), lambda i: (0, 0)),
            pl.BlockSpec((1, 512), lambda i: (0, 0)),
            pl.BlockSpec((512, 256), lambda i: (0, 0)),
        ],
        out_specs=pl.BlockSpec((RPB, 256), lambda i: (i, 0)),
        out_shape=jax.ShapeDtypeStruct((NP, 256), jnp.bfloat16),
    )(acc0, dinv, W1, b1, W2)


def _l2_body(acc_ref, dinv_ref, b2_ref, W3_ref, out_ref):
    a = (acc_ref[0].astype(_f32) + acc_ref[1].astype(_f32))
    d = dinv_ref[...][:, 0:1]
    h = jnp.maximum(a * d + b2_ref[...], 0.0)
    y = jnp.dot(h, W3_ref[...], preferred_element_type=_f32, precision=_HI)
    out_ref[...] = (y * d).astype(jnp.bfloat16)


def _tc_l2(acc1, dinv, b2, W3):
    return pl.pallas_call(
        _l2_body,
        grid=(NB // 2,),
        in_specs=[
            pl.BlockSpec((2, RPB, 256), lambda i: (0, i, 0)),
            pl.BlockSpec((RPB, 16), lambda i: (i, 0)),
            pl.BlockSpec((1, 256), lambda i: (0, 0)),
            pl.BlockSpec((256, 128), lambda i: (0, 0)),
        ],
        out_specs=pl.BlockSpec((RPB, 128), lambda i: (i, 0)),
        out_shape=jax.ShapeDtypeStruct((NP, 128), jnp.bfloat16),
    )(acc1, dinv, b2, W3)


def _l3_body(acc_ref, dinv_ref, b3_ref, out_ref):
    a = acc_ref[0].astype(_f32) + acc_ref[1].astype(_f32)
    d = dinv_ref[...][:, 0:1]
    h = jnp.maximum(a * d + b3_ref[...], 0.0)
    out_ref[...] = (h * d).astype(jnp.bfloat16)


def _tc_l3(acc2, dinv, b3):
    return pl.pallas_call(
        _l3_body,
        grid=(NB // 2,),
        in_specs=[
            pl.BlockSpec((2, RPB, 128), lambda i: (0, i, 0)),
            pl.BlockSpec((RPB, 16), lambda i: (i, 0)),
            pl.BlockSpec((1, 128), lambda i: (0, 0)),
        ],
        out_specs=pl.BlockSpec((RPB, 128), lambda i: (i, 0)),
        out_shape=jax.ShapeDtypeStruct((NP, 128), jnp.bfloat16),
    )(acc2, dinv, b3)


def _head_body(acc_ref, dinv_ref, W4_ref, b4_ref, batch_ref, Wf1_ref,
               bf1_ref, Wf2_ref, bf2_ref, out_ref, sums, cnts):
    i = pl.program_id(0)
    a = acc_ref[0].astype(_f32) + acc_ref[1].astype(_f32)
    d = dinv_ref[...][:, 0:1]
    h = jnp.dot(a * d, W4_ref[...], preferred_element_type=_f32,
                precision=_HI)
    h = jnp.maximum(h + b4_ref[...], 0.0)
    bb = batch_ref[...][:, 0:1]
    oh = (bb == lax.broadcasted_iota(jnp.int32, (RPB, 16), 1)).astype(_f32)
    ssum = lax.dot_general(oh, h, (((0,), (0,)), ((), ())),
                           preferred_element_type=_f32, precision=_HI)
    scnt = lax.dot_general(oh, jnp.ones((RPB, 64), _f32),
                           (((0,), (0,)), ((), ())),
                           preferred_element_type=_f32, precision=_HI)

    @pl.when(i == 0)
    def _():
        sums[...] = ssum
        cnts[...] = scnt

    @pl.when(i > 0)
    def _():
        sums[...] += ssum
        cnts[...] += scnt

    @pl.when(i == NB // 2 - 1)
    def _():
        g = sums[...] / jnp.maximum(cnts[...], 1.0)
        g1 = jnp.dot(g, Wf1_ref[...], preferred_element_type=_f32,
                     precision=_HI)
        g1 = jnp.maximum(g1 + bf1_ref[...], 0.0)
        out_ref[...] = jnp.dot(g1, Wf2_ref[...], preferred_element_type=_f32,
                               precision=_HI) + bf2_ref[...]


def _tc_head(acc3, dinv, W4, b4, batch2d, Wf1, bf1, Wf2, bf2):
    return pl.pallas_call(
        _head_body,
        grid=(NB // 2,),
        in_specs=[
            pl.BlockSpec((2, RPB, 128), lambda i: (0, i, 0)),
            pl.BlockSpec((RPB, 16), lambda i: (i, 0)),
            pl.BlockSpec((128, 64), lambda i: (0, 0)),
            pl.BlockSpec((1, 64), lambda i: (0, 0)),
            pl.BlockSpec((RPB, 16), lambda i: (i, 0)),
            pl.BlockSpec((64, 32), lambda i: (0, 0)),
            pl.BlockSpec((1, 32), lambda i: (0, 0)),
            pl.BlockSpec((32, 10), lambda i: (0, 0)),
            pl.BlockSpec((1, 10), lambda i: (0, 0)),
        ],
        out_specs=pl.BlockSpec((16, 10), lambda i: (0, 0)),
        out_shape=jax.ShapeDtypeStruct((16, 10), _f32),
        scratch_shapes=[
            pltpu.VMEM((16, 64), _f32),
            pltpu.VMEM((16, 64), _f32),
        ],
    )(acc3, dinv, W4, b4, batch2d, Wf1, bf1, Wf2, bf2)


def kernel(x, edge_index, batch, Wc1, bc1, Wc2, bc2, Wc3, bc3, Wc4, bc4,
           Wf1, bf1, Wf2, bf2):
    src = edge_index[0].astype(jnp.int32)
    dst = edge_index[1].astype(jnp.int32)
    pad = N + (jnp.arange(EP - E, dtype=jnp.int32) % (NP - N))
    src_flat = jnp.concatenate([src, pad])
    dst_flat = jnp.concatenate([dst, pad])
    dst_p = dst_flat.reshape(NCH, CH)
    src_p2 = src_flat.reshape(NCH2, CH2)
    dst_p2 = dst_flat.reshape(NCH2, CH2)

    x_pad = jnp.pad(x, ((0, NP - N), (0, 0)))
    batch_p = jnp.concatenate(
        [batch.astype(jnp.int32), jnp.full((NP - N,), 16, jnp.int32)])
    batch2d = jnp.broadcast_to(batch_p[:, None], (NP, 16))

    degp = _deg_kernel(dst_p).reshape(2, NP, 128)
    dinv, ins0 = _tc_prep(x_pad, degp)

    acc0 = _spmm_edge(ins0, src_p2, dst_p2)
    ins1 = _tc_l1(acc0.reshape(2, NP, 128), dinv, Wc1,
                  bc1.reshape(1, 512), Wc2)
    acc1 = _spmm_l2(ins1, src_p2, dst_p2)
    ins2 = _tc_l2(acc1.reshape(2, NP, 256), dinv, bc2.reshape(1, 256), Wc3)
    acc2 = _spmm_edge(ins2, src_p2, dst_p2)
    ins3 = _tc_l3(acc2.reshape(2, NP, 128), dinv, bc3.reshape(1, 128))
    acc3 = _spmm_edge(ins3, src_p2, dst_p2)
    action = _tc_head(acc3.reshape(2, NP, 128), dinv, Wc4,
                      bc4.reshape(1, 64), batch2d, Wf1, bf1.reshape(1, 32),
                      Wf2, bf2.reshape(1, 10))
    return action
